# baseline (speedup 1.0000x reference)
import jax
import jax.numpy as jnp
from jax import lax
from jax.experimental import pallas as pl
from jax.experimental.pallas import tpu as pltpu
from jax.experimental.pallas import tpu_sc as plsc

N = 50000
D = 256
S = 128
DIM = 2 * D
L = 16
NC, NS = 2, 16
NW = NC * NS
NJ = D // L

TPG = 39
SC_ROWS = NW * TPG * L
Y = N - SC_ROWS
TROWS_PT = TPG * L
IDS_LEN = TROWS_PT

GPC = 3
CH = GPC * L
NRING = 2
MAIN_C = TPG // GPC
MAIN_R = (MAIN_C - 1) // NRING

BR = 512
NB = (Y + BR - 1) // BR


def _phase1_body(n_hbm, seg_hbm, pmax_hbm, psum_hbm, pcnt_hbm,
                 ids_v, buf_v, accmax_v, accsum_v, cnt_v,
                 mxc_v, smc_v, cur_s, cntc_s,
                 sem0, sem1, sem2):
    sems = (sem0, sem1, sem2)
    c = lax.axis_index("c")
    s = lax.axis_index("s")
    w = c * NS + s
    row0 = Y + w * TROWS_PT

    def start_chunk(ci, slot):
        r0 = row0 + ci * CH
        pltpu.async_copy(n_hbm.at[pl.ds(r0, CH)],
                         buf_v.at[pl.ds(slot * CH, CH)], sems[slot])

    def wait_chunk(ci, slot):
        r0 = row0 + ci * CH
        pltpu.make_async_copy(n_hbm.at[pl.ds(r0, CH)],
                              buf_v.at[pl.ds(slot * CH, CH)],
                              sems[slot]).wait()

    for slot in range(NRING - 1):
        start_chunk(slot, slot)

    pltpu.sync_copy(seg_hbm.at[pl.ds(row0, IDS_LEN)],
                    ids_v.at[pl.ds(0, IDS_LEN)])

    neg16 = jnp.full((L,), -jnp.inf, jnp.float32)
    zero16 = jnp.zeros((L,), jnp.float32)

    def init_body(i, car):
        cnt_v.at[i][pl.ds(0, L)] = zero16
        return car
    lax.fori_loop(0, S, init_body, 0)
    for j in range(NJ):
        sl = pl.ds(j * L, L)
        mxc_v[sl] = neg16
        smc_v[sl] = zero16
    cur_s[0] = jnp.int32(-1)
    cntc_s[0] = jnp.float32(0.0)

    def process_group(g, brow):
        i0 = g * L
        ids16 = ids_v[pl.ds(i0, L)]
        cur0 = cur_s[0]
        uniform = jnp.logical_and(ids16[0] == cur0, ids16[L - 1] == cur0)

        @pl.when(uniform)
        def _():
            for j in range(NJ):
                sl = pl.ds(j * L, L)
                v = [buf_v.at[brow + r][sl] for r in range(L)]
                m = [jnp.maximum(v[2 * k], v[2 * k + 1]) for k in range(8)]
                m = [jnp.maximum(m[2 * k], m[2 * k + 1]) for k in range(4)]
                m = [jnp.maximum(m[2 * k], m[2 * k + 1]) for k in range(2)]
                gmax = jnp.maximum(m[0], m[1])
                a = [v[2 * k] + v[2 * k + 1] for k in range(8)]
                a = [a[2 * k] + a[2 * k + 1] for k in range(4)]
                a = [a[2 * k] + a[2 * k + 1] for k in range(2)]
                gsum = a[0] + a[1]
                mxc_v[sl] = jnp.maximum(mxc_v[sl], gmax)
                smc_v[sl] = smc_v[sl] + gsum
            cntc_s[0] = cntc_s[0] + float(L)

        @pl.when(jnp.logical_not(uniform))
        def _():
            def row_body(r, carry):
                cur = carry[0]
                cntc = carry[1]
                mx = list(carry[2:2 + NJ])
                sm = list(carry[2 + NJ:])
                sidv = ids_v[pl.ds(i0 + r, L)]
                sid = sidv[0]
                changed = sid != cur
                tgt = jnp.maximum(cur, 0)

                @pl.when(changed)
                def _():
                    for j in range(NJ):
                        sl = pl.ds(j * L, L)
                        accmax_v.at[tgt][sl] = mx[j]
                        accsum_v.at[tgt][sl] = sm[j]
                    cnt_v.at[tgt][pl.ds(0, L)] = (
                        jnp.zeros((L,), jnp.float32) + cntc)

                row = [buf_v.at[brow + r][pl.ds(j * L, L)]
                       for j in range(NJ)]
                mx = [jnp.where(changed, row[j],
                                jnp.maximum(mx[j], row[j]))
                      for j in range(NJ)]
                sm = [jnp.where(changed, row[j], sm[j] + row[j])
                      for j in range(NJ)]
                cntc = jnp.where(changed, jnp.float32(1.0), cntc + 1.0)
                return (sid, cntc, *mx, *sm)

            init = (cur_s[0], cntc_s[0],
                    *[mxc_v[pl.ds(j * L, L)] for j in range(NJ)],
                    *[smc_v[pl.ds(j * L, L)] for j in range(NJ)])
            fin = lax.fori_loop(0, L, row_body, init)
            cur_s[0] = fin[0]
            cntc_s[0] = fin[1]
            for j in range(NJ):
                sl = pl.ds(j * L, L)
                mxc_v[sl] = fin[2 + j]
                smc_v[sl] = fin[2 + NJ + j]

    def round_body(rr, car):
        for k in range(NRING):
            ci = rr * NRING + k

            wait_chunk(ci, k)

            @pl.when(ci + NRING - 1 < MAIN_C)
            def _(ci=ci, k=k):
                start_chunk(ci + NRING - 1, (k + NRING - 1) % NRING)

            def group_body(gi, car2, ci=ci, k=k):
                process_group(ci * GPC + gi, k * CH + gi * L)
                return car2
            lax.fori_loop(0, GPC, group_body, 0)
        return car

    lax.fori_loop(0, MAIN_R, round_body, 0)

    last = MAIN_C - 1
    wait_chunk(last, last % NRING)

    def group_body_last(gi, car2):
        process_group(last * GPC + gi, (last % NRING) * CH + gi * L)
        return car2
    lax.fori_loop(0, GPC, group_body_last, 0)

    tgt = jnp.maximum(cur_s[0], 0)
    for j in range(NJ):
        sl = pl.ds(j * L, L)
        accmax_v.at[tgt][sl] = mxc_v[sl]
        accsum_v.at[tgt][sl] = smc_v[sl]
    cnt_v.at[tgt][pl.ds(0, L)] = jnp.zeros((L,), jnp.float32) + cntc_s[0]

    h1 = pltpu.async_copy(accmax_v, pmax_hbm.at[w], sems[0])
    h2 = pltpu.async_copy(accsum_v, psum_hbm.at[w], sems[1])
    h3 = pltpu.async_copy(cnt_v, pcnt_hbm.at[w], sems[2])
    h1.wait()
    h2.wait()
    h3.wait()


_phase1 = pl.kernel(
    _phase1_body,
    out_type=[
        jax.ShapeDtypeStruct((NW, S, D), jnp.float32),
        jax.ShapeDtypeStruct((NW, S, D), jnp.float32),
        jax.ShapeDtypeStruct((NW, S, L), jnp.float32),
    ],
    mesh=plsc.VectorSubcoreMesh(core_axis_name="c", subcore_axis_name="s",
                                num_cores=NC, num_subcores=NS),
    scratch_types=[
        pltpu.VMEM((IDS_LEN + L,), jnp.int32),
        pltpu.VMEM((NRING * CH, D), jnp.float32),
        pltpu.VMEM((S, D), jnp.float32),
        pltpu.VMEM((S, D), jnp.float32),
        pltpu.VMEM((S, L), jnp.float32),
        pltpu.VMEM((D,), jnp.float32),
        pltpu.VMEM((D,), jnp.float32),
        pltpu.SMEM((1,), jnp.int32),
        pltpu.SMEM((1,), jnp.float32),
        pltpu.SemaphoreType.DMA,
        pltpu.SemaphoreType.DMA,
        pltpu.SemaphoreType.DMA,
    ],
)


def _tcreduce_body(idc_ref, rows_ref, tmax_ref, tsum_ref, tcnt_ref,
                   amax_v, asum_v, acnt_v):
    pid = pl.program_id(0)

    @pl.when(pid == 0)
    def _():
        amax_v[...] = jnp.full((S, D), -jnp.inf, jnp.float32)
        asum_v[...] = jnp.zeros((S, D), jnp.float32)
        acnt_v[...] = jnp.zeros((1, S), jnp.float32)

    rows = rows_ref[...]
    idc = idc_ref[...]
    rix = lax.broadcasted_iota(jnp.int32, (BR, 1), 0) + pid * BR
    valid = rix < Y

    segs = lax.broadcasted_iota(jnp.int32, (1, S), 1)
    oh = jnp.where(valid, (idc == segs).astype(jnp.float32), 0.0)

    asum_v[...] = asum_v[...] + lax.dot_general(
        oh, rows, (((0,), (0,)), ((), ())),
        preferred_element_type=jnp.float32)
    acnt_v[...] = acnt_v[...] + jnp.sum(oh, axis=0, keepdims=True)

    s_lo = idc_ref[0, 0]
    s_hi = idc_ref[BR - 1, 0]

    def seg_body(sg, car):
        msk = jnp.logical_and(idc == sg, valid)
        red = jnp.max(jnp.where(msk, rows, -jnp.inf), axis=0,
                      keepdims=True)
        prev = amax_v[pl.ds(sg, 1), :]
        amax_v[pl.ds(sg, 1), :] = jnp.maximum(prev, red)
        return car
    lax.fori_loop(s_lo, s_hi + 1, seg_body, 0)

    @pl.when(pid == NB - 1)
    def _():
        tmax_ref[...] = amax_v[...]
        tsum_ref[...] = asum_v[...]
        tcnt_ref[...] = acnt_v[...]


def _combine_body(pmax_ref, psum_ref, pcnt_ref, tmax_ref, tsum_ref,
                  tcnt_ref, gain_ref, bias_ref, out_ref):
    alive = pcnt_ref[...][:, :, :1] > 0.0
    m = jnp.max(jnp.where(alive, pmax_ref[...], -jnp.inf), axis=0)
    sm = jnp.sum(jnp.where(alive, psum_ref[...], 0.0), axis=0)
    cnt = jnp.sum(pcnt_ref[...], axis=0)[:, :1]
    m = jnp.maximum(m, tmax_ref[...])
    sm = sm + tsum_ref[...]
    cnt = cnt + tcnt_ref[...].reshape(S, 1)
    mean = sm / jnp.maximum(cnt, 1.0)
    both = jnp.concatenate([m, mean], axis=-1)
    out_ref[...] = both * gain_ref[...] + bias_ref[...]


def kernel(n, segment_ids, gain, bias):
    seg = segment_ids.astype(jnp.int32)
    idc = seg[:NB * BR].reshape(NB * BR, 1)
    pmax, psum, pcnt = _phase1(n, seg)
    tmax, tsum, tcnt = pl.pallas_call(
        _tcreduce_body,
        grid=(NB,),
        in_specs=[
            pl.BlockSpec((BR, 1), lambda i: (i, 0)),
            pl.BlockSpec((BR, D), lambda i: (i, 0)),
        ],
        out_specs=[
            pl.BlockSpec((S, D), lambda i: (0, 0)),
            pl.BlockSpec((S, D), lambda i: (0, 0)),
            pl.BlockSpec((1, S), lambda i: (0, 0)),
        ],
        out_shape=[
            jax.ShapeDtypeStruct((S, D), jnp.float32),
            jax.ShapeDtypeStruct((S, D), jnp.float32),
            jax.ShapeDtypeStruct((1, S), jnp.float32),
        ],
        scratch_shapes=[
            pltpu.VMEM((S, D), jnp.float32),
            pltpu.VMEM((S, D), jnp.float32),
            pltpu.VMEM((1, S), jnp.float32),
        ],
    )(idc, n)
    out = pl.pallas_call(
        _combine_body,
        out_shape=jax.ShapeDtypeStruct((S, DIM), jnp.float32),
    )(pmax, psum, pcnt, tmax, tsum, tcnt,
      gain.reshape(1, DIM), bias.reshape(1, DIM))
    return out


# speedup vs baseline: 1.3377x; 1.3377x over previous
import jax
import jax.numpy as jnp
from jax import lax
from jax.experimental import pallas as pl
from jax.experimental.pallas import tpu as pltpu
from jax.experimental.pallas import tpu_sc as plsc

N = 50000
D = 256
S = 128
DIM = 2 * D
L = 16
NC, NS = 2, 16
NW = NC * NS
NJ = D // L
G = N // L
GQ, GR = divmod(G, NW)
MAX_G = GQ + 1
IDS_LEN = MAX_G * L
N_PAD = ((NW - 1) * GQ + GR) * L + IDS_LEN
GPC = 4
CH = GPC * L
NRING = 2
MAIN_C = GQ // GPC
MAIN_R = MAIN_C // NRING


def _phase1_body(n_hbm, seg_hbm, pmax_hbm, psum_hbm, pcnt_hbm,
                 ids_v, buf_v, accmax_v, accsum_v, cnt_v,
                 mxc_v, smc_v, cur_s, cntc_s,
                 sem0, sem1, sem2):
    sems = (sem0, sem1, sem2)
    c = lax.axis_index("c")
    s = lax.axis_index("s")
    w = c * NS + s
    base_g = w * GQ + jnp.minimum(w, GR)
    ng = GQ + jnp.where(w < GR, 1, 0).astype(jnp.int32)
    row0 = base_g * L

    def start_chunk(ci, slot):
        r0 = (base_g + ci * GPC) * L
        pltpu.async_copy(n_hbm.at[pl.ds(r0, CH)],
                         buf_v.at[pl.ds(slot * CH, CH)], sems[slot])

    for slot in range(NRING - 1):
        start_chunk(slot, slot)

    hids = pltpu.async_copy(seg_hbm.at[pl.ds(row0, IDS_LEN)],
                            ids_v.at[pl.ds(0, IDS_LEN)], sems[2])

    neg16 = jnp.full((L,), -jnp.inf, jnp.float32)
    zero16 = jnp.zeros((L,), jnp.float32)

    def init_body(i, car):
        cnt_v.at[i][pl.ds(0, L)] = zero16
        return car
    lax.fori_loop(0, S, init_body, 0)
    for j in range(NJ):
        sl = pl.ds(j * L, L)
        mxc_v[sl] = neg16
        smc_v[sl] = zero16
    cur_s[0] = jnp.int32(-1)
    cntc_s[0] = jnp.float32(0.0)
    hids.wait()

    def wait_chunk(ci, slot):
        r0 = (base_g + ci * GPC) * L
        pltpu.make_async_copy(n_hbm.at[pl.ds(r0, CH)],
                              buf_v.at[pl.ds(slot * CH, CH)],
                              sems[slot]).wait()

    def process_group(g, brow):
        i0 = g * L
        ids16 = ids_v[pl.ds(i0, L)]
        cur0 = cur_s[0]
        uniform = jnp.logical_and(ids16[0] == cur0, ids16[L - 1] == cur0)

        @pl.when(uniform)
        def _():
            for j in range(NJ):
                sl = pl.ds(j * L, L)
                v = [buf_v.at[brow + r][sl] for r in range(L)]
                m = [jnp.maximum(v[2 * k], v[2 * k + 1]) for k in range(8)]
                m = [jnp.maximum(m[2 * k], m[2 * k + 1]) for k in range(4)]
                m = [jnp.maximum(m[2 * k], m[2 * k + 1]) for k in range(2)]
                gmax = jnp.maximum(m[0], m[1])
                a = [v[2 * k] + v[2 * k + 1] for k in range(8)]
                a = [a[2 * k] + a[2 * k + 1] for k in range(4)]
                a = [a[2 * k] + a[2 * k + 1] for k in range(2)]
                gsum = a[0] + a[1]
                mxc_v[sl] = jnp.maximum(mxc_v[sl], gmax)
                smc_v[sl] = smc_v[sl] + gsum
            cntc_s[0] = cntc_s[0] + float(L)

        @pl.when(jnp.logical_not(uniform))
        def _():
            def row_body(r, carry):
                cur = carry[0]
                cntc = carry[1]
                mx = list(carry[2:2 + NJ])
                sm = list(carry[2 + NJ:])
                sidv = ids_v[pl.ds(i0 + r, L)]
                sid = sidv[0]
                changed = sid != cur
                tgt = jnp.maximum(cur, 0)

                @pl.when(changed)
                def _():
                    for j in range(NJ):
                        sl = pl.ds(j * L, L)
                        accmax_v.at[tgt][sl] = mx[j]
                        accsum_v.at[tgt][sl] = sm[j]
                    cnt_v.at[tgt][pl.ds(0, L)] = (
                        jnp.zeros((L,), jnp.float32) + cntc)

                row = [buf_v.at[brow + r][pl.ds(j * L, L)]
                       for j in range(NJ)]
                mx = [jnp.where(changed, row[j],
                                jnp.maximum(mx[j], row[j]))
                      for j in range(NJ)]
                sm = [jnp.where(changed, row[j], sm[j] + row[j])
                      for j in range(NJ)]
                cntc = jnp.where(changed, jnp.float32(1.0), cntc + 1.0)
                return (sid, cntc, *mx, *sm)

            init = (cur_s[0], cntc_s[0],
                    *[mxc_v[pl.ds(j * L, L)] for j in range(NJ)],
                    *[smc_v[pl.ds(j * L, L)] for j in range(NJ)])
            fin = lax.fori_loop(0, L, row_body, init)
            cur_s[0] = fin[0]
            cntc_s[0] = fin[1]
            for j in range(NJ):
                sl = pl.ds(j * L, L)
                mxc_v[sl] = fin[2 + j]
                smc_v[sl] = fin[2 + NJ + j]

    def round_body(rr, car):
        for k in range(NRING):
            ci = rr * NRING + k

            wait_chunk(ci, k)

            @pl.when(ci + NRING - 1 < MAIN_C)
            def _(ci=ci, k=k):
                start_chunk(ci + NRING - 1, (k + NRING - 1) % NRING)

            def group_body(gi, car2, ci=ci, k=k):
                process_group(ci * GPC + gi, k * CH + gi * L)
                return car2
            lax.fori_loop(0, GPC, group_body, 0)
        return car

    lax.fori_loop(0, MAIN_R, round_body, 0)

    g96 = MAIN_C * GPC
    pltpu.sync_copy(n_hbm.at[pl.ds((base_g + g96) * L, L)],
                    buf_v.at[pl.ds(0, L)])
    process_group(g96, 0)

    @pl.when(ng > g96 + 1)
    def _():
        pltpu.sync_copy(n_hbm.at[pl.ds((base_g + g96 + 1) * L, L)],
                        buf_v.at[pl.ds(0, L)])
        process_group(g96 + 1, 0)

    tgt = jnp.maximum(cur_s[0], 0)
    for j in range(NJ):
        sl = pl.ds(j * L, L)
        accmax_v.at[tgt][sl] = mxc_v[sl]
        accsum_v.at[tgt][sl] = smc_v[sl]
    cnt_v.at[tgt][pl.ds(0, L)] = jnp.zeros((L,), jnp.float32) + cntc_s[0]

    h1 = pltpu.async_copy(accmax_v, pmax_hbm.at[w], sems[0])
    h2 = pltpu.async_copy(accsum_v, psum_hbm.at[w], sems[1])
    h3 = pltpu.async_copy(cnt_v, pcnt_hbm.at[w], sems[2])
    h1.wait()
    h2.wait()
    h3.wait()


_phase1 = pl.kernel(
    _phase1_body,
    out_type=[
        jax.ShapeDtypeStruct((NW, S, D), jnp.float32),
        jax.ShapeDtypeStruct((NW, S, D), jnp.float32),
        jax.ShapeDtypeStruct((NW, S, L), jnp.float32),
    ],
    mesh=plsc.VectorSubcoreMesh(core_axis_name="c", subcore_axis_name="s",
                                num_cores=NC, num_subcores=NS),
    scratch_types=[
        pltpu.VMEM((IDS_LEN + L,), jnp.int32),
        pltpu.VMEM((NRING * CH, D), jnp.float32),
        pltpu.VMEM((S, D), jnp.float32),
        pltpu.VMEM((S, D), jnp.float32),
        pltpu.VMEM((S, L), jnp.float32),
        pltpu.VMEM((D,), jnp.float32),
        pltpu.VMEM((D,), jnp.float32),
        pltpu.SMEM((1,), jnp.int32),
        pltpu.SMEM((1,), jnp.float32),
        pltpu.SemaphoreType.DMA,
        pltpu.SemaphoreType.DMA,
        pltpu.SemaphoreType.DMA,
    ],
)


def _combine_body(pmax_ref, psum_ref, pcnt_ref, gain_ref, bias_ref, out_ref):
    alive = pcnt_ref[...][:, :, :1] > 0.0
    m = jnp.max(jnp.where(alive, pmax_ref[...], -jnp.inf), axis=0)
    sm = jnp.sum(jnp.where(alive, psum_ref[...], 0.0), axis=0)
    cnt = jnp.sum(pcnt_ref[...], axis=0)[:, :1]
    mean = sm / jnp.maximum(cnt, 1.0)
    both = jnp.concatenate([m, mean], axis=-1)
    out_ref[...] = both * gain_ref[...] + bias_ref[...]


def kernel(n, segment_ids, gain, bias):
    seg = segment_ids.astype(jnp.int32)
    seg_pad = jnp.pad(seg, (0, N_PAD - N))
    pmax, psum, pcnt = _phase1(n, seg_pad)
    out = pl.pallas_call(
        _combine_body,
        out_shape=jax.ShapeDtypeStruct((S, DIM), jnp.float32),
    )(pmax, psum, pcnt, gain.reshape(1, DIM), bias.reshape(1, DIM))
    return out
